# baseline (speedup 1.0000x reference)
import jax
import jax.numpy as jnp
from jax import lax
from jax.experimental import pallas as pl
from jax.experimental.pallas import tpu as pltpu
from jax.experimental.pallas import tpu_sc as plsc

N = 10000
E = 320000
D = 128
DE = 16
H = 128

_NC = 2
_NS = 16
_NW = _NC * _NS
_NPAD = 10112
_ROWS_PER_SC = _NPAD // _NS

_CHUNK = 128
_E_PAD = 327680
_CPT = _E_PAD // (_NW * _CHUNK)


def _fill_rows(ref, nrows, ncols, val):
    v = jnp.full((16,), val, jnp.float32)
    nc = ncols // 16

    def body(r, c):
        for j in range(nc):
            ref[r, pl.ds(j * 16, 16)] = v
        return c

    lax.fori_loop(0, nrows, body, 0)


def _make_seg_sum(with_deg):
    out_type = [jax.ShapeDtypeStruct((_NC, _NPAD, D), jnp.float32)]
    if with_deg:
        out_type.append(jax.ShapeDtypeStruct((_NC, _NPAD, D), jnp.float32))
    scratch = [
        pltpu.VMEM((2, _CHUNK), jnp.int32),
        pltpu.VMEM((2, _CHUNK), jnp.int32),
        pltpu.VMEM((2, _CHUNK, D), jnp.float32),
        pltpu.VMEM_SHARED((_NPAD, D), jnp.float32),
        pltpu.SemaphoreType.DMA,
        pltpu.SemaphoreType.DMA,
        pltpu.SemaphoreType.DMA,
    ]

    def body(table, src2d, dst2d, *refs):
        if with_deg:
            aggp, degp, sidx, didx, rows, agg_s, semg, semi0, semi1 = refs
        else:
            aggp, sidx, didx, rows, agg_s, semg, semi0, semi1 = refs
        semi = (semi0, semi1)
        cid = lax.axis_index("c")
        sid = lax.axis_index("s")
        wid = cid * _NS + sid
        r0 = sid * _ROWS_PER_SC
        rem = _ROWS_PER_SC % _CHUNK
        start = wid * _CPT

        def zero_my_slice():
            _fill_rows(rows.at[0], _CHUNK, D, 0.0)
            for z in range(_ROWS_PER_SC // _CHUNK):
                pltpu.sync_copy(rows.at[0],
                                agg_s.at[pl.ds(r0 + z * _CHUNK, _CHUNK)])
            pltpu.sync_copy(rows.at[0, pl.ds(0, rem)],
                            agg_s.at[pl.ds(r0 + _ROWS_PER_SC - rem, rem)])

        def stage_idx_sync(b, j):
            pltpu.sync_copy(src2d.at[pl.ds(start + b, 1)],
                            sidx.at[pl.ds(j, 1)])
            pltpu.sync_copy(dst2d.at[pl.ds(start + b, 1)],
                            didx.at[pl.ds(j, 1)])

        def stage_idx_async(b, j):
            pltpu.async_copy(src2d.at[pl.ds(start + b, 1)],
                             sidx.at[pl.ds(j, 1)], semi[j])
            pltpu.async_copy(dst2d.at[pl.ds(start + b, 1)],
                             didx.at[pl.ds(j, 1)], semi[j])

        def wait_idx(j):
            pltpu.make_async_copy(src2d.at[pl.ds(0, 1)],
                                  sidx.at[pl.ds(j, 1)], semi[j]).wait()
            pltpu.make_async_copy(dst2d.at[pl.ds(0, 1)],
                                  didx.at[pl.ds(j, 1)], semi[j]).wait()

        def start_gather(b, j):
            pltpu.async_copy(table.at[sidx.at[j]], rows.at[j], semg)

        def wait_gather(j):
            pltpu.make_async_copy(table.at[pl.ds(0, _CHUNK)],
                                  rows.at[j], semg).wait()

        zero_my_slice()
        plsc.subcore_barrier()

        stage_idx_sync(0, 0)
        start_gather(0, 0)
        stage_idx_async(1, 1)

        def stepA(i, c):
            for j in (0, 1):
                b = 2 * i + j
                wait_gather(j)
                pltpu.sync_copy(rows.at[j], agg_s.at[didx.at[j]], add=True)

                @pl.when(b + 2 < _CPT)
                def _():
                    stage_idx_async(b + 2, j)

                @pl.when(b + 1 < _CPT)
                def _():
                    wait_idx(1 - j)
                    start_gather(b + 1, 1 - j)
            return c

        lax.fori_loop(0, _CPT // 2, stepA, 0)
        plsc.subcore_barrier()
        pltpu.sync_copy(agg_s.at[pl.ds(r0, _ROWS_PER_SC)],
                        aggp.at[cid, pl.ds(r0, _ROWS_PER_SC)])

        if with_deg:
            zero_my_slice()
            _fill_rows(rows.at[1], _CHUNK, D, 1.0)
            plsc.subcore_barrier()

            pltpu.sync_copy(dst2d.at[pl.ds(start, 1)], didx.at[pl.ds(0, 1)])
            pltpu.async_copy(dst2d.at[pl.ds(start + 1, 1)],
                             didx.at[pl.ds(1, 1)], semi[1])

            def stepB(i, c):
                for j in (0, 1):
                    b = 2 * i + j

                    @pl.when(b >= 1)
                    def _():
                        pltpu.make_async_copy(
                            dst2d.at[pl.ds(0, 1)],
                            didx.at[pl.ds(j, 1)], semi[j]).wait()

                    pltpu.sync_copy(rows.at[1], agg_s.at[didx.at[j]],
                                    add=True)

                    @pl.when(b + 2 < _CPT)
                    def _():
                        pltpu.async_copy(dst2d.at[pl.ds(start + b + 2, 1)],
                                         didx.at[pl.ds(j, 1)], semi[j])
                return c

            lax.fori_loop(0, _CPT // 2, stepB, 0)
            plsc.subcore_barrier()
            pltpu.sync_copy(agg_s.at[pl.ds(r0, _ROWS_PER_SC)],
                            degp.at[cid, pl.ds(r0, _ROWS_PER_SC)])

    mesh = plsc.VectorSubcoreMesh(core_axis_name="c", subcore_axis_name="s")
    return pl.kernel(body, out_type=out_type, mesh=mesh,
                     scratch_types=scratch,
                     name="sc_seg_sum_deg" if with_deg else "sc_seg_sum")


_seg_sum_deg = _make_seg_sum(True)
_seg_sum = _make_seg_sum(False)


def _make_edge_gather():
    out_type = [jax.ShapeDtypeStruct((_E_PAD, D), jnp.float32)]
    scratch = [
        pltpu.VMEM((2, _CHUNK), jnp.int32),
        pltpu.VMEM((2, _CHUNK), jnp.int32),
        pltpu.VMEM((2, _CHUNK, D), jnp.float32),
        pltpu.VMEM((2, _CHUNK, D), jnp.float32),
        pltpu.SemaphoreType.DMA,
        pltpu.SemaphoreType.DMA,
        pltpu.SemaphoreType.DMA,
        pltpu.SemaphoreType.DMA,
    ]

    def body(ps, pd, src2d, dst2d, g_out, sidx, didx, ra, rb,
             semg, semw, semi0, semi1):
        semi = (semi0, semi1)
        cid = lax.axis_index("c")
        sid = lax.axis_index("s")
        wid = cid * _NS + sid
        start = wid * _CPT

        def stage_idx_async(b, j):
            pltpu.async_copy(src2d.at[pl.ds(start + b, 1)],
                             sidx.at[pl.ds(j, 1)], semi[j])
            pltpu.async_copy(dst2d.at[pl.ds(start + b, 1)],
                             didx.at[pl.ds(j, 1)], semi[j])

        def wait_idx(j):
            pltpu.make_async_copy(src2d.at[pl.ds(0, 1)],
                                  sidx.at[pl.ds(j, 1)], semi[j]).wait()
            pltpu.make_async_copy(dst2d.at[pl.ds(0, 1)],
                                  didx.at[pl.ds(j, 1)], semi[j]).wait()

        def start_gathers(b, j):
            pltpu.async_copy(ps.at[sidx.at[j]], ra.at[j], semg)
            pltpu.async_copy(pd.at[didx.at[j]], rb.at[j], semg)

        def wait_gathers(j):
            pltpu.make_async_copy(ps.at[pl.ds(0, _CHUNK)], ra.at[j],
                                  semg).wait()
            pltpu.make_async_copy(pd.at[pl.ds(0, _CHUNK)], rb.at[j],
                                  semg).wait()

        pltpu.sync_copy(src2d.at[pl.ds(start, 1)], sidx.at[pl.ds(0, 1)])
        pltpu.sync_copy(dst2d.at[pl.ds(start, 1)], didx.at[pl.ds(0, 1)])
        start_gathers(0, 0)
        stage_idx_async(1, 1)

        def step(i, c):
            for j in (0, 1):
                b = 2 * i + j
                wait_gathers(j)

                @pl.when(b >= 2)
                def _():
                    pltpu.make_async_copy(
                        ra.at[j], g_out.at[pl.ds(0, _CHUNK)], semw).wait()

                def add_body(r, c2):
                    for q in range(D // 16):
                        sl = pl.ds(q * 16, 16)
                        ra[j, r, sl] = ra[j, r, sl] + rb[j, r, sl]
                    return c2

                lax.fori_loop(0, _CHUNK, add_body, 0)
                pltpu.async_copy(
                    ra.at[j],
                    g_out.at[pl.ds((start + b) * _CHUNK, _CHUNK)], semw)

                @pl.when(b + 2 < _CPT)
                def _():
                    stage_idx_async(b + 2, j)

                @pl.when(b + 1 < _CPT)
                def _():
                    wait_idx(1 - j)
                    start_gathers(b + 1, 1 - j)
            return c

        lax.fori_loop(0, _CPT // 2, step, 0)
        for j in (0, 1):
            pltpu.make_async_copy(ra.at[j], g_out.at[pl.ds(0, _CHUNK)],
                                  semw).wait()

    mesh = plsc.VectorSubcoreMesh(core_axis_name="c", subcore_axis_name="s")
    return pl.kernel(body, out_type=out_type, mesh=mesh,
                     scratch_types=scratch, name="sc_edge_gather")


_edge_gather = _make_edge_gather()


def _dotT(a, w):
    return lax.dot_general(a, w, (((1,), (1,)), ((), ())),
                           preferred_element_type=jnp.float32)


def _sage_body(aggp, degp, x, wl, bl, wr, out):
    a = aggp[...]
    dp = degp[...]
    deg = dp[0, :N, 0:1] + dp[1, :N, 0:1]
    rdeg = 1.0 / jnp.maximum(deg, 1.0)
    mean = (a[0, :N] + a[1, :N]) * rdeg
    h = _dotT(mean, wl[...]) + bl[...] + _dotT(x[...], wr[...])
    out[...] = jnp.maximum(h, 0.0)


_tc_layer1 = pl.pallas_call(
    _sage_body,
    out_shape=jax.ShapeDtypeStruct((N, D), jnp.float32),
)


def _sage2_body(aggp2, degp, h1, wl, bl, wr, wc1s, wc1d, ps_out, pd_out):
    a = aggp2[...]
    dp = degp[...]
    deg = dp[0, :N, 0:1] + dp[1, :N, 0:1]
    rdeg = 1.0 / jnp.maximum(deg, 1.0)
    mean = (a[0, :N] + a[1, :N]) * rdeg
    h = _dotT(mean, wl[...]) + bl[...] + _dotT(h1[...], wr[...])
    h2 = jnp.maximum(h, 0.0)
    zpad = jnp.zeros((_NPAD - N, D), jnp.float32)
    ps_out[...] = jnp.concatenate([_dotT(h2, wc1s[...]), zpad])
    pd_out[...] = jnp.concatenate([_dotT(h2, wc1d[...]), zpad])


_tc_layer2 = pl.pallas_call(
    _sage2_body,
    out_shape=[jax.ShapeDtypeStruct((_NPAD, D), jnp.float32),
               jax.ShapeDtypeStruct((_NPAD, D), jnp.float32)],
)


_EBLK = 16384


def _edge_mlp_body(g, ea, wc1e, bc1, wc2, bc2, wc3, bc3, out):
    z1 = g[...] + _dotT(ea[...], wc1e[...]) + bc1[...]
    z1 = jnp.maximum(z1, 0.0)
    z2 = jnp.maximum(_dotT(z1, wc2[...]) + bc2[...], 0.0)
    lo = lax.dot_general(wc3[...], z2, (((1,), (1,)), ((), ())),
                         preferred_element_type=jnp.float32) + bc3[...]
    out[...] = 1.0 / (1.0 + jnp.exp(-lo))


_tc_edge_mlp = pl.pallas_call(
    _edge_mlp_body,
    grid=(_E_PAD // _EBLK,),
    in_specs=[
        pl.BlockSpec((_EBLK, D), lambda i: (i, 0)),
        pl.BlockSpec((_EBLK, DE), lambda i: (i, 0)),
        pl.BlockSpec((H, DE), lambda i: (0, 0)),
        pl.BlockSpec((1, H), lambda i: (0, 0)),
        pl.BlockSpec((64, H), lambda i: (0, 0)),
        pl.BlockSpec((1, 64), lambda i: (0, 0)),
        pl.BlockSpec((1, 64), lambda i: (0, 0)),
        pl.BlockSpec((1, 1), lambda i: (0, 0)),
    ],
    out_specs=pl.BlockSpec((1, _EBLK), lambda i: (0, i)),
    out_shape=jax.ShapeDtypeStruct((1, _E_PAD), jnp.float32),
)


def kernel(x, edge_index, edge_attr, Wl1, bl1, Wr1, Wl2, bl2, Wr2,
           Wc1, bc1, Wc2, bc2, Wc3, bc3):
    npad_e = _E_PAD - E
    src = jnp.concatenate([edge_index[0], jnp.zeros((npad_e,), jnp.int32)])
    dst = jnp.concatenate([edge_index[1],
                           jnp.full((npad_e,), N, jnp.int32)])
    src2d = src.reshape(_E_PAD // _CHUNK, _CHUNK)
    dst2d = dst.reshape(_E_PAD // _CHUNK, _CHUNK)
    ea = jnp.concatenate([edge_attr, jnp.zeros((npad_e, DE), jnp.float32)])

    aggp, degp = _seg_sum_deg(x, src2d, dst2d)
    h1 = _tc_layer1(aggp, degp, x, Wl1, bl1.reshape(1, H), Wr1)
    aggp2, = _seg_sum(h1, src2d, dst2d)
    ps, pd = _tc_layer2(aggp2, degp, h1, Wl2, bl2.reshape(1, H), Wr2,
                        Wc1[:, :H], Wc1[:, H:2 * H])
    g, = _edge_gather(ps, pd, src2d, dst2d)
    out = _tc_edge_mlp(g, ea, Wc1[:, 2 * H:], bc1.reshape(1, H),
                       Wc2, bc2.reshape(1, 64), Wc3, bc3.reshape(1, 1))
    return out.reshape(-1)[:E]


# speedup vs baseline: 1.0250x; 1.0250x over previous
import jax
import jax.numpy as jnp
from jax import lax
from jax.experimental import pallas as pl
from jax.experimental.pallas import tpu as pltpu
from jax.experimental.pallas import tpu_sc as plsc

N = 10000
E = 320000
D = 128
DE = 16
H = 128

_NC = 2
_NS = 16
_NW = _NC * _NS
_NPAD = 10112
_ROWS_PER_SC = _NPAD // _NS

_CHUNK = 128
_E_PAD = 327680
_CPT = _E_PAD // (_NW * _CHUNK)


def _fill_rows(ref, nrows, ncols, val):
    v = jnp.full((16,), val, jnp.float32)
    nc = ncols // 16

    def body(r, c):
        for j in range(nc):
            ref[r, pl.ds(j * 16, 16)] = v
        return c

    lax.fori_loop(0, nrows, body, 0)


def _make_seg_sum(with_deg):
    out_type = [jax.ShapeDtypeStruct((_NC, _NPAD, D), jnp.float32)]
    if with_deg:
        out_type.append(jax.ShapeDtypeStruct((_NC, _NPAD, D), jnp.float32))
    scratch = [
        pltpu.VMEM((2, _CHUNK), jnp.int32),
        pltpu.VMEM((2, _CHUNK), jnp.int32),
        pltpu.VMEM((2, _CHUNK, D), jnp.float32),
        pltpu.VMEM_SHARED((_NPAD, D), jnp.float32),
        pltpu.SemaphoreType.DMA,
        pltpu.SemaphoreType.DMA,
        pltpu.SemaphoreType.DMA,
    ]

    def body(table, src2d, dst2d, *refs):
        if with_deg:
            aggp, degp, sidx, didx, rows, agg_s, semg, semi0, semi1 = refs
        else:
            aggp, sidx, didx, rows, agg_s, semg, semi0, semi1 = refs
        semi = (semi0, semi1)
        cid = lax.axis_index("c")
        sid = lax.axis_index("s")
        wid = cid * _NS + sid
        r0 = sid * _ROWS_PER_SC
        rem = _ROWS_PER_SC % _CHUNK
        start = wid * _CPT

        def zero_my_slice():
            _fill_rows(rows.at[0], _CHUNK, D, 0.0)
            for z in range(_ROWS_PER_SC // _CHUNK):
                pltpu.sync_copy(rows.at[0],
                                agg_s.at[pl.ds(r0 + z * _CHUNK, _CHUNK)])
            pltpu.sync_copy(rows.at[0, pl.ds(0, rem)],
                            agg_s.at[pl.ds(r0 + _ROWS_PER_SC - rem, rem)])

        def stage_idx_sync(b, j):
            pltpu.sync_copy(src2d.at[pl.ds(start + b, 1)],
                            sidx.at[pl.ds(j, 1)])
            pltpu.sync_copy(dst2d.at[pl.ds(start + b, 1)],
                            didx.at[pl.ds(j, 1)])

        def stage_idx_async(b, j):
            pltpu.async_copy(src2d.at[pl.ds(start + b, 1)],
                             sidx.at[pl.ds(j, 1)], semi[j])
            pltpu.async_copy(dst2d.at[pl.ds(start + b, 1)],
                             didx.at[pl.ds(j, 1)], semi[j])

        def wait_idx(j):
            pltpu.make_async_copy(src2d.at[pl.ds(0, 1)],
                                  sidx.at[pl.ds(j, 1)], semi[j]).wait()
            pltpu.make_async_copy(dst2d.at[pl.ds(0, 1)],
                                  didx.at[pl.ds(j, 1)], semi[j]).wait()

        def start_gather(b, j):
            pltpu.async_copy(table.at[sidx.at[j]], rows.at[j], semg)

        def wait_gather(j):
            pltpu.make_async_copy(table.at[pl.ds(0, _CHUNK)],
                                  rows.at[j], semg).wait()

        zero_my_slice()
        plsc.subcore_barrier()

        stage_idx_sync(0, 0)
        start_gather(0, 0)
        stage_idx_async(1, 1)

        def stepA(i, c):
            for j in (0, 1):
                b = 2 * i + j
                wait_gather(j)
                pltpu.sync_copy(rows.at[j], agg_s.at[didx.at[j]], add=True)

                @pl.when(b + 2 < _CPT)
                def _():
                    stage_idx_async(b + 2, j)

                @pl.when(b + 1 < _CPT)
                def _():
                    wait_idx(1 - j)
                    start_gather(b + 1, 1 - j)
            return c

        lax.fori_loop(0, _CPT // 2, stepA, 0)
        plsc.subcore_barrier()
        pltpu.sync_copy(agg_s.at[pl.ds(r0, _ROWS_PER_SC)],
                        aggp.at[cid, pl.ds(r0, _ROWS_PER_SC)])

        if with_deg:
            zero_my_slice()
            _fill_rows(rows.at[1], _CHUNK, D, 1.0)
            plsc.subcore_barrier()

            pltpu.sync_copy(dst2d.at[pl.ds(start, 1)], didx.at[pl.ds(0, 1)])
            pltpu.async_copy(dst2d.at[pl.ds(start + 1, 1)],
                             didx.at[pl.ds(1, 1)], semi[1])

            def stepB(i, c):
                for j in (0, 1):
                    b = 2 * i + j

                    @pl.when(b >= 1)
                    def _():
                        pltpu.make_async_copy(
                            dst2d.at[pl.ds(0, 1)],
                            didx.at[pl.ds(j, 1)], semi[j]).wait()

                    pltpu.sync_copy(rows.at[1], agg_s.at[didx.at[j]],
                                    add=True)

                    @pl.when(b + 2 < _CPT)
                    def _():
                        pltpu.async_copy(dst2d.at[pl.ds(start + b + 2, 1)],
                                         didx.at[pl.ds(j, 1)], semi[j])
                return c

            lax.fori_loop(0, _CPT // 2, stepB, 0)
            plsc.subcore_barrier()
            pltpu.sync_copy(agg_s.at[pl.ds(r0, _ROWS_PER_SC)],
                            degp.at[cid, pl.ds(r0, _ROWS_PER_SC)])

    mesh = plsc.VectorSubcoreMesh(core_axis_name="c", subcore_axis_name="s")
    return pl.kernel(body, out_type=out_type, mesh=mesh,
                     scratch_types=scratch,
                     name="sc_seg_sum_deg" if with_deg else "sc_seg_sum")


_seg_sum_deg = _make_seg_sum(True)
_seg_sum = _make_seg_sum(False)


def _make_edge_gather():
    out_type = [jax.ShapeDtypeStruct((_E_PAD, D), jnp.float32)]
    scratch = [
        pltpu.VMEM((2, _CHUNK), jnp.int32),
        pltpu.VMEM((2, _CHUNK), jnp.int32),
        pltpu.VMEM((2, _CHUNK, D), jnp.float32),
        pltpu.VMEM((2, _CHUNK, D), jnp.float32),
        pltpu.SemaphoreType.DMA,
        pltpu.SemaphoreType.DMA,
        pltpu.SemaphoreType.DMA,
        pltpu.SemaphoreType.DMA,
    ]

    def body(ps, pd, src2d, dst2d, g_out, sidx, didx, ra, rb,
             semg, semw, semi0, semi1):
        semi = (semi0, semi1)
        cid = lax.axis_index("c")
        sid = lax.axis_index("s")
        wid = cid * _NS + sid
        start = wid * _CPT

        def stage_idx_async(b, j):
            pltpu.async_copy(src2d.at[pl.ds(start + b, 1)],
                             sidx.at[pl.ds(j, 1)], semi[j])
            pltpu.async_copy(dst2d.at[pl.ds(start + b, 1)],
                             didx.at[pl.ds(j, 1)], semi[j])

        def wait_idx(j):
            pltpu.make_async_copy(src2d.at[pl.ds(0, 1)],
                                  sidx.at[pl.ds(j, 1)], semi[j]).wait()
            pltpu.make_async_copy(dst2d.at[pl.ds(0, 1)],
                                  didx.at[pl.ds(j, 1)], semi[j]).wait()

        def start_gathers(b, j):
            pltpu.async_copy(ps.at[sidx.at[j]], ra.at[j], semg)
            pltpu.async_copy(pd.at[didx.at[j]], rb.at[j], semg)

        def wait_gathers(j):
            pltpu.make_async_copy(ps.at[pl.ds(0, _CHUNK)], ra.at[j],
                                  semg).wait()
            pltpu.make_async_copy(pd.at[pl.ds(0, _CHUNK)], rb.at[j],
                                  semg).wait()

        pltpu.sync_copy(src2d.at[pl.ds(start, 1)], sidx.at[pl.ds(0, 1)])
        pltpu.sync_copy(dst2d.at[pl.ds(start, 1)], didx.at[pl.ds(0, 1)])
        start_gathers(0, 0)
        stage_idx_async(1, 1)

        def step(i, c):
            for j in (0, 1):
                b = 2 * i + j
                wait_gathers(j)

                @pl.when(b >= 2)
                def _():
                    pltpu.make_async_copy(
                        ra.at[j], g_out.at[pl.ds(0, _CHUNK)], semw).wait()

                def add_body(r, c2):
                    for q in range(D // 16):
                        sl = pl.ds(q * 16, 16)
                        ra[j, r, sl] = ra[j, r, sl] + rb[j, r, sl]
                    return c2

                lax.fori_loop(0, _CHUNK, add_body, 0)
                pltpu.async_copy(
                    ra.at[j],
                    g_out.at[pl.ds((start + b) * _CHUNK, _CHUNK)], semw)

                @pl.when(b + 2 < _CPT)
                def _():
                    stage_idx_async(b + 2, j)

                @pl.when(b + 1 < _CPT)
                def _():
                    wait_idx(1 - j)
                    start_gathers(b + 1, 1 - j)
            return c

        lax.fori_loop(0, _CPT // 2, step, 0)
        for j in (0, 1):
            pltpu.make_async_copy(ra.at[j], g_out.at[pl.ds(0, _CHUNK)],
                                  semw).wait()

    mesh = plsc.VectorSubcoreMesh(core_axis_name="c", subcore_axis_name="s")
    return pl.kernel(body, out_type=out_type, mesh=mesh,
                     scratch_types=scratch, name="sc_edge_gather")


_edge_gather = _make_edge_gather()


def _dotT(a, w):
    return lax.dot_general(a, w, (((1,), (1,)), ((), ())),
                           preferred_element_type=jnp.float32)


def _sage_body(aggp, degp, x, wl, bl, wr, out):
    a = aggp[...]
    dp = degp[...]
    deg = dp[0, :N, 0:1] + dp[1, :N, 0:1]
    rdeg = 1.0 / jnp.maximum(deg, 1.0)
    mean = (a[0, :N] + a[1, :N]) * rdeg
    h = _dotT(mean, wl[...]) + bl[...] + _dotT(x[...], wr[...])
    out[...] = jnp.maximum(h, 0.0)


_tc_layer1 = pl.pallas_call(
    _sage_body,
    out_shape=jax.ShapeDtypeStruct((N, D), jnp.float32),
)


def _sage2_body(aggp2, degp, h1, wl, bl, wr, wc1s, wc1d, ps_out, pd_out):
    a = aggp2[...]
    dp = degp[...]
    deg = dp[0, :N, 0:1] + dp[1, :N, 0:1]
    rdeg = 1.0 / jnp.maximum(deg, 1.0)
    mean = (a[0, :N] + a[1, :N]) * rdeg
    h = _dotT(mean, wl[...]) + bl[...] + _dotT(h1[...], wr[...])
    h2 = jnp.maximum(h, 0.0)
    zpad = jnp.zeros((_NPAD - N, D), jnp.float32)
    ps_out[...] = jnp.concatenate([_dotT(h2, wc1s[...]), zpad])
    pd_out[...] = jnp.concatenate([_dotT(h2, wc1d[...]), zpad])


_tc_layer2 = pl.pallas_call(
    _sage2_body,
    out_shape=[jax.ShapeDtypeStruct((_NPAD, D), jnp.float32),
               jax.ShapeDtypeStruct((_NPAD, D), jnp.float32)],
)


_EBLK = 16384


def _edge_mlp_body(g, ea, wc1e, bc1, wc2, bc2, wc3, bc3, out):
    z1 = g[...] + _dotT(ea[...], wc1e[...]) + bc1[...]
    z1 = jnp.maximum(z1, 0.0)
    z2 = jnp.maximum(_dotT(z1, wc2[...]) + bc2[...], 0.0)
    lo = lax.dot_general(wc3[...], z2, (((1,), (1,)), ((), ())),
                         preferred_element_type=jnp.float32) + bc3[...]
    out[...] = 1.0 / (1.0 + jnp.exp(-lo))


_tc_edge_mlp = pl.pallas_call(
    _edge_mlp_body,
    grid=(_E_PAD // _EBLK,),
    in_specs=[
        pl.BlockSpec((_EBLK, D), lambda i: (i, 0)),
        pl.BlockSpec((_EBLK, DE), lambda i: (i, 0)),
        pl.BlockSpec((H, DE), lambda i: (0, 0)),
        pl.BlockSpec((1, H), lambda i: (0, 0)),
        pl.BlockSpec((64, H), lambda i: (0, 0)),
        pl.BlockSpec((1, 64), lambda i: (0, 0)),
        pl.BlockSpec((1, 64), lambda i: (0, 0)),
        pl.BlockSpec((1, 1), lambda i: (0, 0)),
    ],
    out_specs=pl.BlockSpec((1, _EBLK), lambda i: (0, i)),
    out_shape=jax.ShapeDtypeStruct((1, _E_PAD), jnp.float32),
)


def kernel(x, edge_index, edge_attr, Wl1, bl1, Wr1, Wl2, bl2, Wr2,
           Wc1, bc1, Wc2, bc2, Wc3, bc3):
    npad_e = _E_PAD - E
    src = jnp.concatenate([edge_index[0], jnp.zeros((npad_e,), jnp.int32)])
    pad_dst = N + (jnp.arange(npad_e, dtype=jnp.int32) % (_NPAD - N))
    dst = jnp.concatenate([edge_index[1], pad_dst])
    src2d = src.reshape(_E_PAD // _CHUNK, _CHUNK)
    dst2d = dst.reshape(_E_PAD // _CHUNK, _CHUNK)
    ea = jnp.concatenate([edge_attr, jnp.zeros((npad_e, DE), jnp.float32)])

    aggp, degp = _seg_sum_deg(x, src2d, dst2d)
    h1 = _tc_layer1(aggp, degp, x, Wl1, bl1.reshape(1, H), Wr1)
    aggp2, = _seg_sum(h1, src2d, dst2d)
    ps, pd = _tc_layer2(aggp2, degp, h1, Wl2, bl2.reshape(1, H), Wr2,
                        Wc1[:, :H], Wc1[:, H:2 * H])
    g, = _edge_gather(ps, pd, src2d, dst2d)
    out = _tc_edge_mlp(g, ea, Wc1[:, 2 * H:], bc1.reshape(1, H),
                       Wc2, bc2.reshape(1, 64), Wc3, bc3.reshape(1, 1))
    return out.reshape(-1)[:E]


# speedup vs baseline: 2.6493x; 2.5847x over previous
import jax
import jax.numpy as jnp
from jax import lax
from jax.experimental import pallas as pl
from jax.experimental.pallas import tpu as pltpu
from jax.experimental.pallas import tpu_sc as plsc

N = 10000
E = 320000
D = 128
DE = 16
H = 128

_NC = 2
_NS = 16
_NW = _NC * _NS
_NPAD = 10112
_ROWS_PER_SC = _NPAD // _NS

_CHUNK = 128
_E_PAD = 327680
_CPT = _E_PAD // (_NW * _CHUNK)


def _fill_rows(ref, nrows, ncols, val):
    v = jnp.full((16,), val, jnp.float32)
    nc = ncols // 16

    def body(r, c):
        for j in range(nc):
            ref[r, pl.ds(j * 16, 16)] = v
        return c

    lax.fori_loop(0, nrows, body, 0)


def _make_seg_sum(with_deg):
    out_type = [jax.ShapeDtypeStruct((_NC, _NPAD, D), jnp.float32)]
    if with_deg:
        out_type.append(jax.ShapeDtypeStruct((_NC, _NPAD, D), jnp.float32))
    scratch = [
        pltpu.VMEM((2, _CHUNK), jnp.int32),
        pltpu.VMEM((2, _CHUNK), jnp.int32),
        pltpu.VMEM((2, _CHUNK, D), jnp.float32),
        pltpu.VMEM_SHARED((_NPAD, D), jnp.float32),
        pltpu.SemaphoreType.DMA,
        pltpu.SemaphoreType.DMA,
        pltpu.SemaphoreType.DMA,
    ]

    def body(table, src2d, dst2d, *refs):
        if with_deg:
            aggp, degp, sidx, didx, rows, agg_s, semg, semi0, semi1 = refs
        else:
            aggp, sidx, didx, rows, agg_s, semg, semi0, semi1 = refs
        semi = (semi0, semi1)
        cid = lax.axis_index("c")
        sid = lax.axis_index("s")
        wid = cid * _NS + sid
        r0 = sid * _ROWS_PER_SC
        rem = _ROWS_PER_SC % _CHUNK
        start = wid * _CPT

        def zero_my_slice():
            _fill_rows(rows.at[0], _CHUNK, D, 0.0)
            for z in range(_ROWS_PER_SC // _CHUNK):
                pltpu.sync_copy(rows.at[0],
                                agg_s.at[pl.ds(r0 + z * _CHUNK, _CHUNK)])
            pltpu.sync_copy(rows.at[0, pl.ds(0, rem)],
                            agg_s.at[pl.ds(r0 + _ROWS_PER_SC - rem, rem)])

        def stage_idx_sync(b, j):
            pltpu.sync_copy(src2d.at[pl.ds(start + b, 1)],
                            sidx.at[pl.ds(j, 1)])
            pltpu.sync_copy(dst2d.at[pl.ds(start + b, 1)],
                            didx.at[pl.ds(j, 1)])

        def stage_idx_async(b, j):
            pltpu.async_copy(src2d.at[pl.ds(start + b, 1)],
                             sidx.at[pl.ds(j, 1)], semi[j])
            pltpu.async_copy(dst2d.at[pl.ds(start + b, 1)],
                             didx.at[pl.ds(j, 1)], semi[j])

        def wait_idx(j):
            pltpu.make_async_copy(src2d.at[pl.ds(0, 1)],
                                  sidx.at[pl.ds(j, 1)], semi[j]).wait()
            pltpu.make_async_copy(dst2d.at[pl.ds(0, 1)],
                                  didx.at[pl.ds(j, 1)], semi[j]).wait()

        def start_gather(b, j):
            pltpu.async_copy(table.at[sidx.at[j]], rows.at[j], semg)

        def wait_gather(j):
            pltpu.make_async_copy(table.at[pl.ds(0, _CHUNK)],
                                  rows.at[j], semg).wait()

        zero_my_slice()
        plsc.subcore_barrier()

        stage_idx_sync(0, 0)
        start_gather(0, 0)
        stage_idx_async(1, 1)

        def stepA(i, c):
            for j in (0, 1):
                b = 2 * i + j
                wait_gather(j)

                @pl.when(b + 1 < _CPT)
                def _():
                    wait_idx(1 - j)
                    start_gather(b + 1, 1 - j)

                pltpu.sync_copy(rows.at[j], agg_s.at[didx.at[j]], add=True)

                @pl.when(b + 2 < _CPT)
                def _():
                    stage_idx_async(b + 2, j)
            return c

        lax.fori_loop(0, _CPT // 2, stepA, 0)
        plsc.subcore_barrier()
        pltpu.sync_copy(agg_s.at[pl.ds(r0, _ROWS_PER_SC)],
                        aggp.at[cid, pl.ds(r0, _ROWS_PER_SC)])

        if with_deg:
            zero_my_slice()
            _fill_rows(rows.at[1], _CHUNK, D, 1.0)
            plsc.subcore_barrier()

            pltpu.sync_copy(dst2d.at[pl.ds(start, 1)], didx.at[pl.ds(0, 1)])
            pltpu.async_copy(dst2d.at[pl.ds(start + 1, 1)],
                             didx.at[pl.ds(1, 1)], semi[1])

            def stepB(i, c):
                for j in (0, 1):
                    b = 2 * i + j

                    @pl.when(b >= 1)
                    def _():
                        pltpu.make_async_copy(
                            dst2d.at[pl.ds(0, 1)],
                            didx.at[pl.ds(j, 1)], semi[j]).wait()

                    pltpu.sync_copy(rows.at[1], agg_s.at[didx.at[j]],
                                    add=True)

                    @pl.when(b + 2 < _CPT)
                    def _():
                        pltpu.async_copy(dst2d.at[pl.ds(start + b + 2, 1)],
                                         didx.at[pl.ds(j, 1)], semi[j])
                return c

            lax.fori_loop(0, _CPT // 2, stepB, 0)
            plsc.subcore_barrier()
            pltpu.sync_copy(agg_s.at[pl.ds(r0, _ROWS_PER_SC)],
                            degp.at[cid, pl.ds(r0, _ROWS_PER_SC)])

    mesh = plsc.VectorSubcoreMesh(core_axis_name="c", subcore_axis_name="s")
    return pl.kernel(body, out_type=out_type, mesh=mesh,
                     scratch_types=scratch,
                     name="sc_seg_sum_deg" if with_deg else "sc_seg_sum")


_seg_sum_deg = _make_seg_sum(True)
_seg_sum = _make_seg_sum(False)


def _make_edge_gather():
    out_type = [jax.ShapeDtypeStruct((_E_PAD, D), jnp.float32)]
    scratch = [
        pltpu.VMEM((2, _CHUNK), jnp.int32),
        pltpu.VMEM((2, _CHUNK), jnp.int32),
        pltpu.VMEM((2, _CHUNK, D), jnp.float32),
        pltpu.VMEM((2, _CHUNK, D), jnp.float32),
        pltpu.SemaphoreType.DMA,
        pltpu.SemaphoreType.DMA,
        pltpu.SemaphoreType.DMA,
        pltpu.SemaphoreType.DMA,
    ]

    def body(ps, pd, src2d, dst2d, g_out, sidx, didx, ra, rb,
             semg, semw, semi0, semi1):
        semi = (semi0, semi1)
        cid = lax.axis_index("c")
        sid = lax.axis_index("s")
        wid = cid * _NS + sid
        start = wid * _CPT

        def stage_idx_async(b, j):
            pltpu.async_copy(src2d.at[pl.ds(start + b, 1)],
                             sidx.at[pl.ds(j, 1)], semi[j])
            pltpu.async_copy(dst2d.at[pl.ds(start + b, 1)],
                             didx.at[pl.ds(j, 1)], semi[j])

        def wait_idx(j):
            pltpu.make_async_copy(src2d.at[pl.ds(0, 1)],
                                  sidx.at[pl.ds(j, 1)], semi[j]).wait()
            pltpu.make_async_copy(dst2d.at[pl.ds(0, 1)],
                                  didx.at[pl.ds(j, 1)], semi[j]).wait()

        def start_gathers(b, j):
            pltpu.async_copy(ps.at[sidx.at[j]], ra.at[j], semg)
            pltpu.async_copy(pd.at[didx.at[j]], rb.at[j], semg)

        def wait_gathers(j):
            pltpu.make_async_copy(ps.at[pl.ds(0, _CHUNK)], ra.at[j],
                                  semg).wait()
            pltpu.make_async_copy(pd.at[pl.ds(0, _CHUNK)], rb.at[j],
                                  semg).wait()

        pltpu.sync_copy(src2d.at[pl.ds(start, 1)], sidx.at[pl.ds(0, 1)])
        pltpu.sync_copy(dst2d.at[pl.ds(start, 1)], didx.at[pl.ds(0, 1)])
        start_gathers(0, 0)
        stage_idx_async(1, 1)

        def step(i, c):
            for j in (0, 1):
                b = 2 * i + j
                wait_gathers(j)

                @pl.when(b + 1 < _CPT)
                def _():
                    wait_idx(1 - j)

                    @pl.when(b >= 1)
                    def _():
                        pltpu.make_async_copy(
                            ra.at[1 - j], g_out.at[pl.ds(0, _CHUNK)],
                            semw).wait()

                    start_gathers(b + 1, 1 - j)

                def add_body(r, c2):
                    for q in range(D // 16):
                        sl = pl.ds(q * 16, 16)
                        ra[j, r, sl] = ra[j, r, sl] + rb[j, r, sl]
                    return c2

                lax.fori_loop(0, _CHUNK, add_body, 0)
                pltpu.async_copy(
                    ra.at[j],
                    g_out.at[pl.ds((start + b) * _CHUNK, _CHUNK)], semw)

                @pl.when(b + 2 < _CPT)
                def _():
                    stage_idx_async(b + 2, j)
            return c

        lax.fori_loop(0, _CPT // 2, step, 0)
        for j in (0, 1):
            pltpu.make_async_copy(ra.at[j], g_out.at[pl.ds(0, _CHUNK)],
                                  semw).wait()

    mesh = plsc.VectorSubcoreMesh(core_axis_name="c", subcore_axis_name="s")
    return pl.kernel(body, out_type=out_type, mesh=mesh,
                     scratch_types=scratch, name="sc_edge_gather")


_edge_gather = _make_edge_gather()


def _dotT(a, w):
    return lax.dot_general(a, w, (((1,), (1,)), ((), ())),
                           preferred_element_type=jnp.float32)


def _sage_body(aggp, degp, x, wl, bl, wr, out):
    a = aggp[...]
    dp = degp[...]
    deg = dp[0, :N, 0:1] + dp[1, :N, 0:1]
    rdeg = 1.0 / jnp.maximum(deg, 1.0)
    mean = (a[0, :N] + a[1, :N]) * rdeg
    h = _dotT(mean, wl[...]) + bl[...] + _dotT(x[...], wr[...])
    out[...] = jnp.maximum(h, 0.0)


_tc_layer1 = pl.pallas_call(
    _sage_body,
    out_shape=jax.ShapeDtypeStruct((N, D), jnp.float32),
)


def _sage2_body(aggp2, degp, h1, wl, bl, wr, wc1s, wc1d, ps_out, pd_out):
    a = aggp2[...]
    dp = degp[...]
    deg = dp[0, :N, 0:1] + dp[1, :N, 0:1]
    rdeg = 1.0 / jnp.maximum(deg, 1.0)
    mean = (a[0, :N] + a[1, :N]) * rdeg
    h = _dotT(mean, wl[...]) + bl[...] + _dotT(h1[...], wr[...])
    h2 = jnp.maximum(h, 0.0)
    zpad = jnp.zeros((_NPAD - N, D), jnp.float32)
    ps_out[...] = jnp.concatenate([_dotT(h2, wc1s[...]), zpad])
    pd_out[...] = jnp.concatenate([_dotT(h2, wc1d[...]), zpad])


_tc_layer2 = pl.pallas_call(
    _sage2_body,
    out_shape=[jax.ShapeDtypeStruct((_NPAD, D), jnp.float32),
               jax.ShapeDtypeStruct((_NPAD, D), jnp.float32)],
)


_EBLK = 16384


def _edge_mlp_body(g, ea, wc1e, bc1, wc2, bc2, wc3, bc3, out):
    z1 = g[...] + _dotT(ea[...], wc1e[...]) + bc1[...]
    z1 = jnp.maximum(z1, 0.0)
    z2 = jnp.maximum(_dotT(z1, wc2[...]) + bc2[...], 0.0)
    lo = lax.dot_general(wc3[...], z2, (((1,), (1,)), ((), ())),
                         preferred_element_type=jnp.float32) + bc3[...]
    out[...] = 1.0 / (1.0 + jnp.exp(-lo))


_tc_edge_mlp = pl.pallas_call(
    _edge_mlp_body,
    grid=(_E_PAD // _EBLK,),
    in_specs=[
        pl.BlockSpec((_EBLK, D), lambda i: (i, 0)),
        pl.BlockSpec((_EBLK, DE), lambda i: (i, 0)),
        pl.BlockSpec((H, DE), lambda i: (0, 0)),
        pl.BlockSpec((1, H), lambda i: (0, 0)),
        pl.BlockSpec((64, H), lambda i: (0, 0)),
        pl.BlockSpec((1, 64), lambda i: (0, 0)),
        pl.BlockSpec((1, 64), lambda i: (0, 0)),
        pl.BlockSpec((1, 1), lambda i: (0, 0)),
    ],
    out_specs=pl.BlockSpec((1, _EBLK), lambda i: (0, i)),
    out_shape=jax.ShapeDtypeStruct((1, _E_PAD), jnp.float32),
)


def kernel(x, edge_index, edge_attr, Wl1, bl1, Wr1, Wl2, bl2, Wr2,
           Wc1, bc1, Wc2, bc2, Wc3, bc3):
    npad_e = _E_PAD - E
    pad_src = jnp.arange(npad_e, dtype=jnp.int32) % N
    pad_dst = N + (jnp.arange(npad_e, dtype=jnp.int32) % (_NPAD - N))
    src = jnp.concatenate([edge_index[0], pad_src])
    dst = jnp.concatenate([edge_index[1], pad_dst])
    src2d = src.reshape(_E_PAD // _CHUNK, _CHUNK)
    dst2d = dst.reshape(_E_PAD // _CHUNK, _CHUNK)
    ea = jnp.concatenate([edge_attr, jnp.zeros((npad_e, DE), jnp.float32)])

    aggp, degp = _seg_sum_deg(x, src2d, dst2d)
    h1 = _tc_layer1(aggp, degp, x, Wl1, bl1.reshape(1, H), Wr1)
    aggp2, = _seg_sum(h1, src2d, dst2d)
    ps, pd = _tc_layer2(aggp2, degp, h1, Wl2, bl2.reshape(1, H), Wr2,
                        Wc1[:, :H], Wc1[:, H:2 * H])
    g, = _edge_gather(ps, pd, src2d, dst2d)
    out = _tc_edge_mlp(g, ea, Wc1[:, 2 * H:], bc1.reshape(1, H),
                       Wc2, bc2.reshape(1, 64), Wc3, bc3.reshape(1, 1))
    return out.reshape(-1)[:E]


# speedup vs baseline: 2.6532x; 1.0014x over previous
import jax
import jax.numpy as jnp
from jax import lax
from jax.experimental import pallas as pl
from jax.experimental.pallas import tpu as pltpu
from jax.experimental.pallas import tpu_sc as plsc

N = 10000
E = 320000
D = 128
DE = 16
H = 128

_NC = 2
_NS = 16
_NW = _NC * _NS
_NPAD = 10112
_ROWS_PER_SC = _NPAD // _NS

_CHUNK = 128
_E_PAD = 327680
_CPT = _E_PAD // (_NW * _CHUNK)


def _fill_rows(ref, nrows, ncols, val):
    v = jnp.full((16,), val, jnp.float32)
    nc = ncols // 16

    def body(r, c):
        for j in range(nc):
            ref[r, pl.ds(j * 16, 16)] = v
        return c

    lax.fori_loop(0, nrows, body, 0)


def _make_seg_sum(with_deg):
    out_type = [jax.ShapeDtypeStruct((_NC, _NPAD, D), jnp.float32)]
    if with_deg:
        out_type.append(jax.ShapeDtypeStruct((_NC, _NPAD, D), jnp.float32))
    scratch = [
        pltpu.VMEM((2, _CHUNK), jnp.int32),
        pltpu.VMEM((2, _CHUNK), jnp.int32),
        pltpu.VMEM((2, _CHUNK, D), jnp.float32),
        pltpu.VMEM_SHARED((_NPAD, D), jnp.float32),
        pltpu.SemaphoreType.DMA,
        pltpu.SemaphoreType.DMA,
        pltpu.SemaphoreType.DMA,
    ]

    def body(table, src2d, dst2d, *refs):
        if with_deg:
            aggp, degp, sidx, didx, rows, agg_s, semg, semi0, semi1 = refs
        else:
            aggp, sidx, didx, rows, agg_s, semg, semi0, semi1 = refs
        semi = (semi0, semi1)
        cid = lax.axis_index("c")
        sid = lax.axis_index("s")
        wid = cid * _NS + sid
        r0 = sid * _ROWS_PER_SC
        rem = _ROWS_PER_SC % _CHUNK
        start = wid * _CPT

        def zero_my_slice():
            _fill_rows(rows.at[0], _CHUNK, D, 0.0)
            for z in range(_ROWS_PER_SC // _CHUNK):
                pltpu.sync_copy(rows.at[0],
                                agg_s.at[pl.ds(r0 + z * _CHUNK, _CHUNK)])
            pltpu.sync_copy(rows.at[0, pl.ds(0, rem)],
                            agg_s.at[pl.ds(r0 + _ROWS_PER_SC - rem, rem)])

        def stage_idx_sync(b, j):
            pltpu.sync_copy(src2d.at[pl.ds(start + b, 1)],
                            sidx.at[pl.ds(j, 1)])
            pltpu.sync_copy(dst2d.at[pl.ds(start + b, 1)],
                            didx.at[pl.ds(j, 1)])

        def stage_idx_async(b, j):
            pltpu.async_copy(src2d.at[pl.ds(start + b, 1)],
                             sidx.at[pl.ds(j, 1)], semi[j])
            pltpu.async_copy(dst2d.at[pl.ds(start + b, 1)],
                             didx.at[pl.ds(j, 1)], semi[j])

        def wait_idx(j):
            pltpu.make_async_copy(src2d.at[pl.ds(0, 1)],
                                  sidx.at[pl.ds(j, 1)], semi[j]).wait()
            pltpu.make_async_copy(dst2d.at[pl.ds(0, 1)],
                                  didx.at[pl.ds(j, 1)], semi[j]).wait()

        def start_gather(b, j):
            pltpu.async_copy(table.at[sidx.at[j]], rows.at[j], semg)

        def wait_gather(j):
            pltpu.make_async_copy(table.at[pl.ds(0, _CHUNK)],
                                  rows.at[j], semg).wait()

        zero_my_slice()
        plsc.subcore_barrier()

        stage_idx_sync(0, 0)
        start_gather(0, 0)
        stage_idx_async(1, 1)

        def stepA(i, c):
            for j in (0, 1):
                b = 2 * i + j
                wait_gather(j)

                @pl.when(b + 1 < _CPT)
                def _():
                    wait_idx(1 - j)
                    start_gather(b + 1, 1 - j)

                pltpu.sync_copy(rows.at[j], agg_s.at[didx.at[j]], add=True)

                @pl.when(b + 2 < _CPT)
                def _():
                    stage_idx_async(b + 2, j)
            return c

        lax.fori_loop(0, _CPT // 2, stepA, 0)
        plsc.subcore_barrier()
        pltpu.sync_copy(agg_s.at[pl.ds(r0, _ROWS_PER_SC)],
                        aggp.at[cid, pl.ds(r0, _ROWS_PER_SC)])

        if with_deg:
            zero_my_slice()
            _fill_rows(rows.at[1], _CHUNK, D, 1.0)
            plsc.subcore_barrier()

            pltpu.sync_copy(dst2d.at[pl.ds(start, 1)], didx.at[pl.ds(0, 1)])
            pltpu.async_copy(dst2d.at[pl.ds(start + 1, 1)],
                             didx.at[pl.ds(1, 1)], semi[1])

            def stepB(i, c):
                for j in (0, 1):
                    b = 2 * i + j

                    @pl.when(b >= 1)
                    def _():
                        pltpu.make_async_copy(
                            dst2d.at[pl.ds(0, 1)],
                            didx.at[pl.ds(j, 1)], semi[j]).wait()

                    pltpu.sync_copy(rows.at[1], agg_s.at[didx.at[j]],
                                    add=True)

                    @pl.when(b + 2 < _CPT)
                    def _():
                        pltpu.async_copy(dst2d.at[pl.ds(start + b + 2, 1)],
                                         didx.at[pl.ds(j, 1)], semi[j])
                return c

            lax.fori_loop(0, _CPT // 2, stepB, 0)
            plsc.subcore_barrier()
            pltpu.sync_copy(agg_s.at[pl.ds(r0, _ROWS_PER_SC)],
                            degp.at[cid, pl.ds(r0, _ROWS_PER_SC)])

    mesh = plsc.VectorSubcoreMesh(core_axis_name="c", subcore_axis_name="s")
    return pl.kernel(body, out_type=out_type, mesh=mesh,
                     scratch_types=scratch,
                     name="sc_seg_sum_deg" if with_deg else "sc_seg_sum")


_seg_sum_deg = _make_seg_sum(True)
_seg_sum = _make_seg_sum(False)


def _make_edge_gather():
    out_type = [jax.ShapeDtypeStruct((_E_PAD, D), jnp.float32)]
    scratch = [
        pltpu.VMEM((2, _CHUNK), jnp.int32),
        pltpu.VMEM((2, _CHUNK), jnp.int32),
        pltpu.VMEM((2, _CHUNK, D), jnp.float32),
        pltpu.VMEM((2, _CHUNK, D), jnp.float32),
        pltpu.SemaphoreType.DMA,
        pltpu.SemaphoreType.DMA,
        pltpu.SemaphoreType.DMA,
        pltpu.SemaphoreType.DMA,
    ]

    def body(ps, pd, src2d, dst2d, g_out, sidx, didx, ra, rb,
             semg, semw, semi0, semi1):
        semi = (semi0, semi1)
        cid = lax.axis_index("c")
        sid = lax.axis_index("s")
        wid = cid * _NS + sid
        start = wid * _CPT

        def stage_idx_async(b, j):
            pltpu.async_copy(src2d.at[pl.ds(start + b, 1)],
                             sidx.at[pl.ds(j, 1)], semi[j])
            pltpu.async_copy(dst2d.at[pl.ds(start + b, 1)],
                             didx.at[pl.ds(j, 1)], semi[j])

        def wait_idx(j):
            pltpu.make_async_copy(src2d.at[pl.ds(0, 1)],
                                  sidx.at[pl.ds(j, 1)], semi[j]).wait()
            pltpu.make_async_copy(dst2d.at[pl.ds(0, 1)],
                                  didx.at[pl.ds(j, 1)], semi[j]).wait()

        def start_gathers(b, j):
            pltpu.async_copy(ps.at[sidx.at[j]], ra.at[j], semg)
            pltpu.async_copy(pd.at[didx.at[j]], rb.at[j], semg)

        def wait_gathers(j):
            pltpu.make_async_copy(ps.at[pl.ds(0, _CHUNK)], ra.at[j],
                                  semg).wait()
            pltpu.make_async_copy(pd.at[pl.ds(0, _CHUNK)], rb.at[j],
                                  semg).wait()

        pltpu.sync_copy(src2d.at[pl.ds(start, 1)], sidx.at[pl.ds(0, 1)])
        pltpu.sync_copy(dst2d.at[pl.ds(start, 1)], didx.at[pl.ds(0, 1)])
        start_gathers(0, 0)
        stage_idx_async(1, 1)

        def step(i, c):
            for j in (0, 1):
                b = 2 * i + j
                wait_gathers(j)

                @pl.when(b + 1 < _CPT)
                def _():
                    wait_idx(1 - j)

                    @pl.when(b >= 1)
                    def _():
                        pltpu.make_async_copy(
                            ra.at[1 - j], g_out.at[pl.ds(0, _CHUNK)],
                            semw).wait()

                    start_gathers(b + 1, 1 - j)

                @plsc.parallel_loop(0, _CHUNK, unroll=4)
                def _(r):
                    for q in range(D // 16):
                        sl = pl.ds(q * 16, 16)
                        ra[j, r, sl] = ra[j, r, sl] + rb[j, r, sl]
                pltpu.async_copy(
                    ra.at[j],
                    g_out.at[pl.ds((start + b) * _CHUNK, _CHUNK)], semw)

                @pl.when(b + 2 < _CPT)
                def _():
                    stage_idx_async(b + 2, j)
            return c

        lax.fori_loop(0, _CPT // 2, step, 0)
        for j in (0, 1):
            pltpu.make_async_copy(ra.at[j], g_out.at[pl.ds(0, _CHUNK)],
                                  semw).wait()

    mesh = plsc.VectorSubcoreMesh(core_axis_name="c", subcore_axis_name="s")
    return pl.kernel(body, out_type=out_type, mesh=mesh,
                     scratch_types=scratch, name="sc_edge_gather")


_edge_gather = _make_edge_gather()


def _dotT(a, w):
    return lax.dot_general(a, w, (((1,), (1,)), ((), ())),
                           preferred_element_type=jnp.float32)


def _sage_body(aggp, degp, x, wl, bl, wr, out):
    a = aggp[...]
    dp = degp[...]
    deg = dp[0, :N, 0:1] + dp[1, :N, 0:1]
    rdeg = 1.0 / jnp.maximum(deg, 1.0)
    mean = (a[0, :N] + a[1, :N]) * rdeg
    h = _dotT(mean, wl[...]) + bl[...] + _dotT(x[...], wr[...])
    out[...] = jnp.maximum(h, 0.0)


_tc_layer1 = pl.pallas_call(
    _sage_body,
    out_shape=jax.ShapeDtypeStruct((N, D), jnp.float32),
)


def _sage2_body(aggp2, degp, h1, wl, bl, wr, wc1s, wc1d, ps_out, pd_out):
    a = aggp2[...]
    dp = degp[...]
    deg = dp[0, :N, 0:1] + dp[1, :N, 0:1]
    rdeg = 1.0 / jnp.maximum(deg, 1.0)
    mean = (a[0, :N] + a[1, :N]) * rdeg
    h = _dotT(mean, wl[...]) + bl[...] + _dotT(h1[...], wr[...])
    h2 = jnp.maximum(h, 0.0)
    zpad = jnp.zeros((_NPAD - N, D), jnp.float32)
    ps_out[...] = jnp.concatenate([_dotT(h2, wc1s[...]), zpad])
    pd_out[...] = jnp.concatenate([_dotT(h2, wc1d[...]), zpad])


_tc_layer2 = pl.pallas_call(
    _sage2_body,
    out_shape=[jax.ShapeDtypeStruct((_NPAD, D), jnp.float32),
               jax.ShapeDtypeStruct((_NPAD, D), jnp.float32)],
)


_EBLK = 16384


def _edge_mlp_body(g, ea, wc1e, bc1, wc2, bc2, wc3, bc3, out):
    z1 = g[...] + _dotT(ea[...], wc1e[...]) + bc1[...]
    z1 = jnp.maximum(z1, 0.0)
    z2 = jnp.maximum(_dotT(z1, wc2[...]) + bc2[...], 0.0)
    lo = lax.dot_general(wc3[...], z2, (((1,), (1,)), ((), ())),
                         preferred_element_type=jnp.float32) + bc3[...]
    out[...] = 1.0 / (1.0 + jnp.exp(-lo))


_tc_edge_mlp = pl.pallas_call(
    _edge_mlp_body,
    grid=(_E_PAD // _EBLK,),
    in_specs=[
        pl.BlockSpec((_EBLK, D), lambda i: (i, 0)),
        pl.BlockSpec((_EBLK, DE), lambda i: (i, 0)),
        pl.BlockSpec((H, DE), lambda i: (0, 0)),
        pl.BlockSpec((1, H), lambda i: (0, 0)),
        pl.BlockSpec((64, H), lambda i: (0, 0)),
        pl.BlockSpec((1, 64), lambda i: (0, 0)),
        pl.BlockSpec((1, 64), lambda i: (0, 0)),
        pl.BlockSpec((1, 1), lambda i: (0, 0)),
    ],
    out_specs=pl.BlockSpec((1, _EBLK), lambda i: (0, i)),
    out_shape=jax.ShapeDtypeStruct((1, _E_PAD), jnp.float32),
)


def kernel(x, edge_index, edge_attr, Wl1, bl1, Wr1, Wl2, bl2, Wr2,
           Wc1, bc1, Wc2, bc2, Wc3, bc3):
    npad_e = _E_PAD - E
    pad_src = jnp.arange(npad_e, dtype=jnp.int32) % N
    pad_dst = N + (jnp.arange(npad_e, dtype=jnp.int32) % (_NPAD - N))
    src = jnp.concatenate([edge_index[0], pad_src])
    dst = jnp.concatenate([edge_index[1], pad_dst])
    src2d = src.reshape(_E_PAD // _CHUNK, _CHUNK)
    dst2d = dst.reshape(_E_PAD // _CHUNK, _CHUNK)
    ea = jnp.concatenate([edge_attr, jnp.zeros((npad_e, DE), jnp.float32)])

    aggp, degp = _seg_sum_deg(x, src2d, dst2d)
    h1 = _tc_layer1(aggp, degp, x, Wl1, bl1.reshape(1, H), Wr1)
    aggp2, = _seg_sum(h1, src2d, dst2d)
    ps, pd = _tc_layer2(aggp2, degp, h1, Wl2, bl2.reshape(1, H), Wr2,
                        Wc1[:, :H], Wc1[:, H:2 * H])
    g, = _edge_gather(ps, pd, src2d, dst2d)
    out = _tc_edge_mlp(g, ea, Wc1[:, 2 * H:], bc1.reshape(1, H),
                       Wc2, bc2.reshape(1, 64), Wc3, bc3.reshape(1, 1))
    return out.reshape(-1)[:E]


# speedup vs baseline: 2.6534x; 1.0001x over previous
import jax
import jax.numpy as jnp
from jax import lax
from jax.experimental import pallas as pl
from jax.experimental.pallas import tpu as pltpu
from jax.experimental.pallas import tpu_sc as plsc

N = 10000
E = 320000
D = 128
DE = 16
H = 128

_NC = 2
_NS = 16
_NW = _NC * _NS
_NPAD = 10112
_ROWS_PER_SC = _NPAD // _NS

_CHUNK = 128
_E_PAD = 327680
_CPT = _E_PAD // (_NW * _CHUNK)


def _fill_rows(ref, nrows, ncols, val):
    v = jnp.full((16,), val, jnp.float32)
    nc = ncols // 16

    def body(r, c):
        for j in range(nc):
            ref[r, pl.ds(j * 16, 16)] = v
        return c

    lax.fori_loop(0, nrows, body, 0)


def _make_seg_sum(with_deg):
    out_type = [jax.ShapeDtypeStruct((_NC, _NPAD, D), jnp.float32)]
    if with_deg:
        out_type.append(jax.ShapeDtypeStruct((_NC, _NPAD, D), jnp.float32))
    scratch = [
        pltpu.VMEM((2, _CHUNK), jnp.int32),
        pltpu.VMEM((2, _CHUNK), jnp.int32),
        pltpu.VMEM((2, _CHUNK, D), jnp.float32),
        pltpu.VMEM_SHARED((_NPAD, D), jnp.float32),
        pltpu.SemaphoreType.DMA,
        pltpu.SemaphoreType.DMA,
        pltpu.SemaphoreType.DMA,
    ]

    def body(table, src2d, dst2d, *refs):
        if with_deg:
            aggp, degp, sidx, didx, rows, agg_s, semg, semi0, semi1 = refs
        else:
            aggp, sidx, didx, rows, agg_s, semg, semi0, semi1 = refs
        semi = (semi0, semi1)
        cid = lax.axis_index("c")
        sid = lax.axis_index("s")
        wid = cid * _NS + sid
        r0 = sid * _ROWS_PER_SC
        rem = _ROWS_PER_SC % _CHUNK
        start = wid * _CPT

        def zero_my_slice():
            _fill_rows(rows.at[0], _CHUNK, D, 0.0)
            for z in range(_ROWS_PER_SC // _CHUNK):
                pltpu.sync_copy(rows.at[0],
                                agg_s.at[pl.ds(r0 + z * _CHUNK, _CHUNK)])
            pltpu.sync_copy(rows.at[0, pl.ds(0, rem)],
                            agg_s.at[pl.ds(r0 + _ROWS_PER_SC - rem, rem)])

        def stage_idx_sync(b, j):
            pltpu.sync_copy(src2d.at[pl.ds(start + b, 1)],
                            sidx.at[pl.ds(j, 1)])
            pltpu.sync_copy(dst2d.at[pl.ds(start + b, 1)],
                            didx.at[pl.ds(j, 1)])

        def stage_idx_async(b, j):
            pltpu.async_copy(src2d.at[pl.ds(start + b, 1)],
                             sidx.at[pl.ds(j, 1)], semi[j])
            pltpu.async_copy(dst2d.at[pl.ds(start + b, 1)],
                             didx.at[pl.ds(j, 1)], semi[j])

        def wait_idx(j):
            pltpu.make_async_copy(src2d.at[pl.ds(0, 1)],
                                  sidx.at[pl.ds(j, 1)], semi[j]).wait()
            pltpu.make_async_copy(dst2d.at[pl.ds(0, 1)],
                                  didx.at[pl.ds(j, 1)], semi[j]).wait()

        def start_gather(b, j):
            pltpu.async_copy(table.at[sidx.at[j]], rows.at[j], semg)

        def wait_gather(j):
            pltpu.make_async_copy(table.at[pl.ds(0, _CHUNK)],
                                  rows.at[j], semg).wait()

        zero_my_slice()
        plsc.subcore_barrier()

        stage_idx_sync(0, 0)
        start_gather(0, 0)
        stage_idx_async(1, 1)

        def stepA(i, c):
            for j in (0, 1):
                b = 2 * i + j
                wait_gather(j)

                @pl.when(b + 1 < _CPT)
                def _():
                    wait_idx(1 - j)
                    start_gather(b + 1, 1 - j)

                pltpu.sync_copy(rows.at[j], agg_s.at[didx.at[j]], add=True)

                @pl.when(b + 2 < _CPT)
                def _():
                    stage_idx_async(b + 2, j)
            return c

        lax.fori_loop(0, _CPT // 2, stepA, 0)
        plsc.subcore_barrier()
        pltpu.sync_copy(agg_s.at[pl.ds(r0, _ROWS_PER_SC)],
                        aggp.at[cid, pl.ds(r0, _ROWS_PER_SC)])

        if with_deg:
            zero_my_slice()
            _fill_rows(rows.at[1], _CHUNK, D, 1.0)
            plsc.subcore_barrier()

            pltpu.sync_copy(dst2d.at[pl.ds(start, 1)], didx.at[pl.ds(0, 1)])
            pltpu.async_copy(dst2d.at[pl.ds(start + 1, 1)],
                             didx.at[pl.ds(1, 1)], semi[1])

            def stepB(i, c):
                for j in (0, 1):
                    b = 2 * i + j

                    @pl.when(b >= 1)
                    def _():
                        pltpu.make_async_copy(
                            dst2d.at[pl.ds(0, 1)],
                            didx.at[pl.ds(j, 1)], semi[j]).wait()

                    pltpu.sync_copy(rows.at[1], agg_s.at[didx.at[j]],
                                    add=True)

                    @pl.when(b + 2 < _CPT)
                    def _():
                        pltpu.async_copy(dst2d.at[pl.ds(start + b + 2, 1)],
                                         didx.at[pl.ds(j, 1)], semi[j])
                return c

            lax.fori_loop(0, _CPT // 2, stepB, 0)
            plsc.subcore_barrier()
            pltpu.sync_copy(agg_s.at[pl.ds(r0, _ROWS_PER_SC)],
                            degp.at[cid, pl.ds(r0, _ROWS_PER_SC)])

    mesh = plsc.VectorSubcoreMesh(core_axis_name="c", subcore_axis_name="s")
    return pl.kernel(body, out_type=out_type, mesh=mesh,
                     scratch_types=scratch,
                     name="sc_seg_sum_deg" if with_deg else "sc_seg_sum")


_seg_sum_deg = _make_seg_sum(True)
_seg_sum = _make_seg_sum(False)


def _make_edge_gather(half):
    n_ch = _E_PAD // _CHUNK // 2
    cpt = n_ch // _NW
    chunk0 = half * n_ch
    out_type = [jax.ShapeDtypeStruct((_E_PAD // 2, D), jnp.float32)]
    scratch = [
        pltpu.VMEM((2, _CHUNK), jnp.int32),
        pltpu.VMEM((2, _CHUNK), jnp.int32),
        pltpu.VMEM((2, _CHUNK, D), jnp.float32),
        pltpu.VMEM((2, _CHUNK, D), jnp.float32),
        pltpu.SemaphoreType.DMA,
        pltpu.SemaphoreType.DMA,
        pltpu.SemaphoreType.DMA,
        pltpu.SemaphoreType.DMA,
    ]

    def body(ps, pd, src2d, dst2d, g_out, sidx, didx, ra, rb,
             semg, semw, semi0, semi1):
        semi = (semi0, semi1)
        cid = lax.axis_index("c")
        sid = lax.axis_index("s")
        wid = cid * _NS + sid
        lstart = wid * cpt
        start = chunk0 + lstart

        def stage_idx_async(b, j):
            pltpu.async_copy(src2d.at[pl.ds(start + b, 1)],
                             sidx.at[pl.ds(j, 1)], semi[j])
            pltpu.async_copy(dst2d.at[pl.ds(start + b, 1)],
                             didx.at[pl.ds(j, 1)], semi[j])

        def wait_idx(j):
            pltpu.make_async_copy(src2d.at[pl.ds(0, 1)],
                                  sidx.at[pl.ds(j, 1)], semi[j]).wait()
            pltpu.make_async_copy(dst2d.at[pl.ds(0, 1)],
                                  didx.at[pl.ds(j, 1)], semi[j]).wait()

        def start_gathers(b, j):
            pltpu.async_copy(ps.at[sidx.at[j]], ra.at[j], semg)
            pltpu.async_copy(pd.at[didx.at[j]], rb.at[j], semg)

        def wait_gathers(j):
            pltpu.make_async_copy(ps.at[pl.ds(0, _CHUNK)], ra.at[j],
                                  semg).wait()
            pltpu.make_async_copy(pd.at[pl.ds(0, _CHUNK)], rb.at[j],
                                  semg).wait()

        pltpu.sync_copy(src2d.at[pl.ds(start, 1)], sidx.at[pl.ds(0, 1)])
        pltpu.sync_copy(dst2d.at[pl.ds(start, 1)], didx.at[pl.ds(0, 1)])
        start_gathers(0, 0)
        stage_idx_async(1, 1)

        def step(i, c):
            for j in (0, 1):
                b = 2 * i + j
                wait_gathers(j)

                @pl.when(b + 1 < cpt)
                def _():
                    wait_idx(1 - j)

                    @pl.when(b >= 1)
                    def _():
                        pltpu.make_async_copy(
                            ra.at[1 - j], g_out.at[pl.ds(0, _CHUNK)],
                            semw).wait()

                    start_gathers(b + 1, 1 - j)

                @plsc.parallel_loop(0, _CHUNK, unroll=4)
                def _(r):
                    for q in range(D // 16):
                        sl = pl.ds(q * 16, 16)
                        ra[j, r, sl] = ra[j, r, sl] + rb[j, r, sl]
                pltpu.async_copy(
                    ra.at[j],
                    g_out.at[pl.ds((lstart + b) * _CHUNK, _CHUNK)], semw)

                @pl.when(b + 2 < cpt)
                def _():
                    stage_idx_async(b + 2, j)
            return c

        lax.fori_loop(0, cpt // 2, step, 0)
        for j in (0, 1):
            pltpu.make_async_copy(ra.at[j], g_out.at[pl.ds(0, _CHUNK)],
                                  semw).wait()

    mesh = plsc.VectorSubcoreMesh(core_axis_name="c", subcore_axis_name="s")
    return pl.kernel(body, out_type=out_type, mesh=mesh,
                     scratch_types=scratch, name="sc_edge_gather%d" % half)


_edge_gather0 = _make_edge_gather(0)
_edge_gather1 = _make_edge_gather(1)


def _dotT(a, w):
    return lax.dot_general(a, w, (((1,), (1,)), ((), ())),
                           preferred_element_type=jnp.float32)


def _sage_body(aggp, degp, x, wl, bl, wr, out):
    a = aggp[...]
    dp = degp[...]
    deg = dp[0, :N, 0:1] + dp[1, :N, 0:1]
    rdeg = 1.0 / jnp.maximum(deg, 1.0)
    mean = (a[0, :N] + a[1, :N]) * rdeg
    h = _dotT(mean, wl[...]) + bl[...] + _dotT(x[...], wr[...])
    out[...] = jnp.maximum(h, 0.0)


_tc_layer1 = pl.pallas_call(
    _sage_body,
    out_shape=jax.ShapeDtypeStruct((N, D), jnp.float32),
)


def _sage2_body(aggp2, degp, h1, wl, bl, wr, wc1s, wc1d, ps_out, pd_out):
    a = aggp2[...]
    dp = degp[...]
    deg = dp[0, :N, 0:1] + dp[1, :N, 0:1]
    rdeg = 1.0 / jnp.maximum(deg, 1.0)
    mean = (a[0, :N] + a[1, :N]) * rdeg
    h = _dotT(mean, wl[...]) + bl[...] + _dotT(h1[...], wr[...])
    h2 = jnp.maximum(h, 0.0)
    zpad = jnp.zeros((_NPAD - N, D), jnp.float32)
    ps_out[...] = jnp.concatenate([_dotT(h2, wc1s[...]), zpad])
    pd_out[...] = jnp.concatenate([_dotT(h2, wc1d[...]), zpad])


_tc_layer2 = pl.pallas_call(
    _sage2_body,
    out_shape=[jax.ShapeDtypeStruct((_NPAD, D), jnp.float32),
               jax.ShapeDtypeStruct((_NPAD, D), jnp.float32)],
)


_EBLK = 16384
_EHALF = _E_PAD // 2


def _edge_mlp_body(g, ea, wc1e, bc1, wc2, bc2, wc3, bc3, out):
    z1 = g[...] + _dotT(ea[...], wc1e[...]) + bc1[...]
    z1 = jnp.maximum(z1, 0.0)
    z2 = jnp.maximum(_dotT(z1, wc2[...]) + bc2[...], 0.0)
    lo = lax.dot_general(wc3[...], z2, (((1,), (1,)), ((), ())),
                         preferred_element_type=jnp.float32) + bc3[...]
    out[...] = 1.0 / (1.0 + jnp.exp(-lo))


def _make_edge_mlp(half):
    off = half * (_EHALF // _EBLK)
    return pl.pallas_call(
        _edge_mlp_body,
        grid=(_EHALF // _EBLK,),
        in_specs=[
            pl.BlockSpec((_EBLK, D), lambda i: (i, 0)),
            pl.BlockSpec((_EBLK, DE), lambda i: (i + off, 0)),
            pl.BlockSpec((H, DE), lambda i: (0, 0)),
            pl.BlockSpec((1, H), lambda i: (0, 0)),
            pl.BlockSpec((64, H), lambda i: (0, 0)),
            pl.BlockSpec((1, 64), lambda i: (0, 0)),
            pl.BlockSpec((1, 64), lambda i: (0, 0)),
            pl.BlockSpec((1, 1), lambda i: (0, 0)),
        ],
        out_specs=pl.BlockSpec((1, _EBLK), lambda i: (0, i)),
        out_shape=jax.ShapeDtypeStruct((1, _EHALF), jnp.float32),
    )


_tc_edge_mlp0 = _make_edge_mlp(0)
_tc_edge_mlp1 = _make_edge_mlp(1)


def kernel(x, edge_index, edge_attr, Wl1, bl1, Wr1, Wl2, bl2, Wr2,
           Wc1, bc1, Wc2, bc2, Wc3, bc3):
    npad_e = _E_PAD - E
    pad_src = jnp.arange(npad_e, dtype=jnp.int32) % N
    pad_dst = N + (jnp.arange(npad_e, dtype=jnp.int32) % (_NPAD - N))
    src = jnp.concatenate([edge_index[0], pad_src])
    dst = jnp.concatenate([edge_index[1], pad_dst])
    src2d = src.reshape(_E_PAD // _CHUNK, _CHUNK)
    dst2d = dst.reshape(_E_PAD // _CHUNK, _CHUNK)
    ea = jnp.concatenate([edge_attr, jnp.zeros((npad_e, DE), jnp.float32)])

    aggp, degp = _seg_sum_deg(x, src2d, dst2d)
    h1 = _tc_layer1(aggp, degp, x, Wl1, bl1.reshape(1, H), Wr1)
    aggp2, = _seg_sum(h1, src2d, dst2d)
    ps, pd = _tc_layer2(aggp2, degp, h1, Wl2, bl2.reshape(1, H), Wr2,
                        Wc1[:, :H], Wc1[:, H:2 * H])
    g0, = _edge_gather0(ps, pd, src2d, dst2d)
    g1, = _edge_gather1(ps, pd, src2d, dst2d)
    mlp_args = (Wc1[:, 2 * H:], bc1.reshape(1, H),
                Wc2, bc2.reshape(1, 64), Wc3, bc3.reshape(1, 1))
    out0 = _tc_edge_mlp0(g0, ea, *mlp_args)
    out1 = _tc_edge_mlp1(g1, ea, *mlp_args)
    out = jnp.concatenate([out0, out1], axis=1)
    return out.reshape(-1)[:E]


# speedup vs baseline: 2.7633x; 1.0414x over previous
import jax
import jax.numpy as jnp
from jax import lax
from jax.experimental import pallas as pl
from jax.experimental.pallas import tpu as pltpu
from jax.experimental.pallas import tpu_sc as plsc

N = 10000
E = 320000
D = 128
DE = 16
H = 128

_NC = 2
_NS = 16
_NW = _NC * _NS
_NPAD = 10112
_ROWS_PER_SC = _NPAD // _NS

_CHUNK = 128
_E_PAD = 327680
_CPT = _E_PAD // (_NW * _CHUNK)


def _fill_rows(ref, nrows, ncols, val):
    v = jnp.full((16,), val, jnp.float32)
    nc = ncols // 16

    def body(r, c):
        for j in range(nc):
            ref[r, pl.ds(j * 16, 16)] = v
        return c

    lax.fori_loop(0, nrows, body, 0)


def _make_seg_sum(with_deg):
    out_type = [jax.ShapeDtypeStruct((_NC, _NPAD, D), jnp.float32)]
    if with_deg:
        out_type.append(jax.ShapeDtypeStruct((_NC, _NPAD, D), jnp.float32))
    scratch = [
        pltpu.VMEM((2, _CHUNK), jnp.int32),
        pltpu.VMEM((2, _CHUNK), jnp.int32),
        pltpu.VMEM((2, _CHUNK, D), jnp.float32),
        pltpu.VMEM_SHARED((_NPAD, D), jnp.float32),
        pltpu.SemaphoreType.DMA,
        pltpu.SemaphoreType.DMA,
        pltpu.SemaphoreType.DMA,
    ]

    def body(table, src2d, dst2d, *refs):
        if with_deg:
            aggp, degp, sidx, didx, rows, agg_s, semg, semi0, semi1 = refs
        else:
            aggp, sidx, didx, rows, agg_s, semg, semi0, semi1 = refs
        semi = (semi0, semi1)
        cid = lax.axis_index("c")
        sid = lax.axis_index("s")
        wid = cid * _NS + sid
        r0 = sid * _ROWS_PER_SC
        rem = _ROWS_PER_SC % _CHUNK
        start = wid * _CPT

        def zero_my_slice():
            _fill_rows(rows.at[0], _CHUNK, D, 0.0)
            for z in range(_ROWS_PER_SC // _CHUNK):
                pltpu.sync_copy(rows.at[0],
                                agg_s.at[pl.ds(r0 + z * _CHUNK, _CHUNK)])
            pltpu.sync_copy(rows.at[0, pl.ds(0, rem)],
                            agg_s.at[pl.ds(r0 + _ROWS_PER_SC - rem, rem)])

        def stage_idx_sync(b, j):
            pltpu.sync_copy(src2d.at[pl.ds(start + b, 1)],
                            sidx.at[pl.ds(j, 1)])
            pltpu.sync_copy(dst2d.at[pl.ds(start + b, 1)],
                            didx.at[pl.ds(j, 1)])

        def stage_idx_async(b, j):
            pltpu.async_copy(src2d.at[pl.ds(start + b, 1)],
                             sidx.at[pl.ds(j, 1)], semi[j])
            pltpu.async_copy(dst2d.at[pl.ds(start + b, 1)],
                             didx.at[pl.ds(j, 1)], semi[j])

        def wait_idx(j):
            pltpu.make_async_copy(src2d.at[pl.ds(0, 1)],
                                  sidx.at[pl.ds(j, 1)], semi[j]).wait()
            pltpu.make_async_copy(dst2d.at[pl.ds(0, 1)],
                                  didx.at[pl.ds(j, 1)], semi[j]).wait()

        def start_gather(b, j):
            pltpu.async_copy(table.at[sidx.at[j]], rows.at[j], semg)

        def wait_gather(j):
            pltpu.make_async_copy(table.at[pl.ds(0, _CHUNK)],
                                  rows.at[j], semg).wait()

        zero_my_slice()
        plsc.subcore_barrier()

        stage_idx_sync(0, 0)
        start_gather(0, 0)
        stage_idx_async(1, 1)

        def stepA(i, c):
            for j in (0, 1):
                b = 2 * i + j
                wait_gather(j)

                @pl.when(b + 1 < _CPT)
                def _():
                    wait_idx(1 - j)
                    start_gather(b + 1, 1 - j)

                pltpu.sync_copy(rows.at[j], agg_s.at[didx.at[j]], add=True)

                @pl.when(b + 2 < _CPT)
                def _():
                    stage_idx_async(b + 2, j)
            return c

        lax.fori_loop(0, _CPT // 2, stepA, 0)
        plsc.subcore_barrier()
        pltpu.sync_copy(agg_s.at[pl.ds(r0, _ROWS_PER_SC)],
                        aggp.at[cid, pl.ds(r0, _ROWS_PER_SC)])

        if with_deg:
            zero_my_slice()
            _fill_rows(rows.at[1], _CHUNK, D, 1.0)
            plsc.subcore_barrier()

            pltpu.sync_copy(dst2d.at[pl.ds(start, 1)], didx.at[pl.ds(0, 1)])
            pltpu.async_copy(dst2d.at[pl.ds(start + 1, 1)],
                             didx.at[pl.ds(1, 1)], semi[1])

            def stepB(i, c):
                for j in (0, 1):
                    b = 2 * i + j

                    @pl.when(b >= 1)
                    def _():
                        pltpu.make_async_copy(
                            dst2d.at[pl.ds(0, 1)],
                            didx.at[pl.ds(j, 1)], semi[j]).wait()

                    pltpu.sync_copy(rows.at[1], agg_s.at[didx.at[j]],
                                    add=True)

                    @pl.when(b + 2 < _CPT)
                    def _():
                        pltpu.async_copy(dst2d.at[pl.ds(start + b + 2, 1)],
                                         didx.at[pl.ds(j, 1)], semi[j])
                return c

            lax.fori_loop(0, _CPT // 2, stepB, 0)
            plsc.subcore_barrier()
            pltpu.sync_copy(agg_s.at[pl.ds(r0, _ROWS_PER_SC)],
                            degp.at[cid, pl.ds(r0, _ROWS_PER_SC)])

    mesh = plsc.VectorSubcoreMesh(core_axis_name="c", subcore_axis_name="s")
    return pl.kernel(body, out_type=out_type, mesh=mesh,
                     scratch_types=scratch,
                     name="sc_seg_sum_deg" if with_deg else "sc_seg_sum")


_seg_sum_deg = _make_seg_sum(True)
_seg_sum = _make_seg_sum(False)


def _make_edge_gather(half):
    n_ch = _E_PAD // _CHUNK // 2
    cpt = n_ch // _NW
    chunk0 = half * n_ch
    out_type = [jax.ShapeDtypeStruct((_E_PAD // 2, D), jnp.float32)]
    scratch = [
        pltpu.VMEM((2, _CHUNK), jnp.int32),
        pltpu.VMEM((2, _CHUNK), jnp.int32),
        pltpu.VMEM((2, _CHUNK, D), jnp.float32),
        pltpu.VMEM((2, _CHUNK, D), jnp.float32),
        pltpu.SemaphoreType.DMA,
        pltpu.SemaphoreType.DMA,
        pltpu.SemaphoreType.DMA,
        pltpu.SemaphoreType.DMA,
    ]

    def body(ps, pd, src2d, dst2d, g_out, sidx, didx, ra, rb,
             semg, semw, semi0, semi1):
        semi = (semi0, semi1)
        cid = lax.axis_index("c")
        sid = lax.axis_index("s")
        wid = cid * _NS + sid
        lstart = wid * cpt
        start = chunk0 + lstart

        def stage_idx_async(b, j):
            pltpu.async_copy(src2d.at[pl.ds(start + b, 1)],
                             sidx.at[pl.ds(j, 1)], semi[j])
            pltpu.async_copy(dst2d.at[pl.ds(start + b, 1)],
                             didx.at[pl.ds(j, 1)], semi[j])

        def wait_idx(j):
            pltpu.make_async_copy(src2d.at[pl.ds(0, 1)],
                                  sidx.at[pl.ds(j, 1)], semi[j]).wait()
            pltpu.make_async_copy(dst2d.at[pl.ds(0, 1)],
                                  didx.at[pl.ds(j, 1)], semi[j]).wait()

        def start_gathers(b, j):
            pltpu.async_copy(ps.at[sidx.at[j]], ra.at[j], semg)
            pltpu.async_copy(pd.at[didx.at[j]], rb.at[j], semg)

        def wait_gathers(j):
            pltpu.make_async_copy(ps.at[pl.ds(0, _CHUNK)], ra.at[j],
                                  semg).wait()
            pltpu.make_async_copy(pd.at[pl.ds(0, _CHUNK)], rb.at[j],
                                  semg).wait()

        pltpu.sync_copy(src2d.at[pl.ds(start, 1)], sidx.at[pl.ds(0, 1)])
        pltpu.sync_copy(dst2d.at[pl.ds(start, 1)], didx.at[pl.ds(0, 1)])
        start_gathers(0, 0)
        stage_idx_async(1, 1)

        def step(i, c):
            for j in (0, 1):
                b = 2 * i + j
                wait_gathers(j)

                @pl.when(b + 1 < cpt)
                def _():
                    wait_idx(1 - j)

                    @pl.when(b >= 1)
                    def _():
                        pltpu.make_async_copy(
                            ra.at[1 - j], g_out.at[pl.ds(0, _CHUNK)],
                            semw).wait()

                    start_gathers(b + 1, 1 - j)

                @plsc.parallel_loop(0, _CHUNK, unroll=4)
                def _(r):
                    for q in range(D // 16):
                        sl = pl.ds(q * 16, 16)
                        ra[j, r, sl] = ra[j, r, sl] + rb[j, r, sl]
                pltpu.async_copy(
                    ra.at[j],
                    g_out.at[pl.ds((lstart + b) * _CHUNK, _CHUNK)], semw)

                @pl.when(b + 2 < cpt)
                def _():
                    stage_idx_async(b + 2, j)
            return c

        lax.fori_loop(0, cpt // 2, step, 0)
        for j in (0, 1):
            pltpu.make_async_copy(ra.at[j], g_out.at[pl.ds(0, _CHUNK)],
                                  semw).wait()

    mesh = plsc.VectorSubcoreMesh(core_axis_name="c", subcore_axis_name="s")
    return pl.kernel(body, out_type=out_type, mesh=mesh,
                     scratch_types=scratch, name="sc_edge_gather%d" % half)


_edge_gather0 = _make_edge_gather(0)
_edge_gather1 = _make_edge_gather(1)


def _dotT(a, w):
    return lax.dot_general(a, w, (((1,), (1,)), ((), ())),
                           preferred_element_type=jnp.float32)


def _sage_body(aggp, degp, x, wl, bl, wr, out):
    a = aggp[...]
    dp = degp[...]
    deg = dp[0, :N, 0:1] + dp[1, :N, 0:1]
    rdeg = 1.0 / jnp.maximum(deg, 1.0)
    mean = (a[0, :N] + a[1, :N]) * rdeg
    h = _dotT(mean, wl[...]) + bl[...] + _dotT(x[...], wr[...])
    out[...] = jnp.maximum(h, 0.0)


_tc_layer1 = pl.pallas_call(
    _sage_body,
    out_shape=jax.ShapeDtypeStruct((N, D), jnp.float32),
)


def _sage2_body(aggp2, degp, h1, wl, bl, wr, wc1s, wc1d, ps_out, pd_out):
    a = aggp2[...]
    dp = degp[...]
    deg = dp[0, :N, 0:1] + dp[1, :N, 0:1]
    rdeg = 1.0 / jnp.maximum(deg, 1.0)
    mean = (a[0, :N] + a[1, :N]) * rdeg
    h = _dotT(mean, wl[...]) + bl[...] + _dotT(h1[...], wr[...])
    h2 = jnp.maximum(h, 0.0)
    zpad = jnp.zeros((_NPAD - N, D), jnp.float32)
    ps_out[...] = jnp.concatenate([_dotT(h2, wc1s[...]), zpad])
    pd_out[...] = jnp.concatenate([_dotT(h2, wc1d[...]), zpad])


_tc_layer2 = pl.pallas_call(
    _sage2_body,
    out_shape=[jax.ShapeDtypeStruct((_NPAD, D), jnp.float32),
               jax.ShapeDtypeStruct((_NPAD, D), jnp.float32)],
)


_EBLK = 2560
_EHALF = _E_PAD // 2


def _edge_mlp_body(g, ea, wc1e, bc1, wc2, bc2, wc3, bc3, out):
    z1 = g[...] + _dotT(ea[...], wc1e[...]) + bc1[...]
    z1 = jnp.maximum(z1, 0.0)
    z2 = jnp.maximum(_dotT(z1, wc2[...]) + bc2[...], 0.0)
    lo = lax.dot_general(wc3[...], z2, (((1,), (1,)), ((), ())),
                         preferred_element_type=jnp.float32) + bc3[...]
    out[...] = 1.0 / (1.0 + jnp.exp(-lo))


def _make_edge_mlp(half):
    off = half * (_EHALF // _EBLK)
    n_real = min(E - half * _EHALF, _EHALF)
    grid = n_real // _EBLK
    return pl.pallas_call(
        _edge_mlp_body,
        grid=(grid,),
        in_specs=[
            pl.BlockSpec((_EBLK, D), lambda i: (i, 0)),
            pl.BlockSpec((_EBLK, DE), lambda i: (i + off, 0)),
            pl.BlockSpec((H, DE), lambda i: (0, 0)),
            pl.BlockSpec((1, H), lambda i: (0, 0)),
            pl.BlockSpec((64, H), lambda i: (0, 0)),
            pl.BlockSpec((1, 64), lambda i: (0, 0)),
            pl.BlockSpec((1, 64), lambda i: (0, 0)),
            pl.BlockSpec((1, 1), lambda i: (0, 0)),
        ],
        out_specs=pl.BlockSpec((1, _EBLK), lambda i: (0, i)),
        out_shape=jax.ShapeDtypeStruct((1, n_real), jnp.float32),
    )


_tc_edge_mlp0 = _make_edge_mlp(0)
_tc_edge_mlp1 = _make_edge_mlp(1)


def kernel(x, edge_index, edge_attr, Wl1, bl1, Wr1, Wl2, bl2, Wr2,
           Wc1, bc1, Wc2, bc2, Wc3, bc3):
    npad_e = _E_PAD - E
    pad_src = jnp.arange(npad_e, dtype=jnp.int32) % N
    pad_dst = N + (jnp.arange(npad_e, dtype=jnp.int32) % (_NPAD - N))
    src = jnp.concatenate([edge_index[0], pad_src])
    dst = jnp.concatenate([edge_index[1], pad_dst])
    src2d = src.reshape(_E_PAD // _CHUNK, _CHUNK)
    dst2d = dst.reshape(_E_PAD // _CHUNK, _CHUNK)

    aggp, degp = _seg_sum_deg(x, src2d, dst2d)
    h1 = _tc_layer1(aggp, degp, x, Wl1, bl1.reshape(1, H), Wr1)
    aggp2, = _seg_sum(h1, src2d, dst2d)
    ps, pd = _tc_layer2(aggp2, degp, h1, Wl2, bl2.reshape(1, H), Wr2,
                        Wc1[:, :H], Wc1[:, H:2 * H])
    g0, = _edge_gather0(ps, pd, src2d, dst2d)
    g1, = _edge_gather1(ps, pd, src2d, dst2d)
    mlp_args = (Wc1[:, 2 * H:], bc1.reshape(1, H),
                Wc2, bc2.reshape(1, 64), Wc3, bc3.reshape(1, 1))
    out0 = _tc_edge_mlp0(g0, edge_attr, *mlp_args)
    out1 = _tc_edge_mlp1(g1, edge_attr, *mlp_args)
    return jnp.concatenate([out0, out1], axis=1).reshape(-1)
